# Initial kernel scaffold; baseline (speedup 1.0000x reference)
import jax
import jax.numpy as jnp
from jax.experimental import pallas as pl


def kernel(xyz):
    raise NotImplementedError("write your pallas kernel here")


# speedup vs baseline: 8.7445x; 8.7445x over previous
import functools

import jax
import jax.numpy as jnp
from jax import lax
from jax.experimental import pallas as pl
from jax.experimental.pallas import tpu as pltpu
from jax.experimental.pallas import tpu_sc as plsc

B, N, NPOINT, NSAMPLE = 4, 8192, 1024, 32
RADIUS2 = 0.2 * 0.2


def _fps_body(xyz_ref, out_ref):
    x3 = xyz_ref[...]
    lane = lax.broadcasted_iota(jnp.int32, (B, N), 1)

    def body(i, carry):
        dists, far = carry
        onehot = (lane == far)
        cent = jnp.sum(jnp.where(onehot[:, None, :], x3, 0.0), axis=2)
        out_ref[pl.ds(i, 1)] = cent[None]
        diff = x3 - cent[:, :, None]
        d = jnp.sum(diff * diff, axis=1)
        dists = jnp.minimum(dists, d)
        m = jnp.max(dists, axis=1, keepdims=True)
        far = jnp.min(jnp.where(dists == m, lane, N), axis=1,
                      keepdims=True).astype(jnp.int32)
        return dists, far

    dists0 = jnp.full((B, N), 1e10, dtype=jnp.float32)
    far0 = jnp.zeros((B, 1), dtype=jnp.int32)
    lax.fori_loop(0, NPOINT, body, (dists0, far0))


def _run_fps(xyz_t, interpret=False):
    return pl.pallas_call(
        _fps_body,
        out_shape=jax.ShapeDtypeStruct((NPOINT, B, 3), jnp.float32),
        interpret=interpret,
    )(xyz_t)


QT = 256
NC = 512
NCHUNK = N // NC

def _ball_body(xyz_ref, newx_ref, idx_ref):
    nx = newx_ref[0]
    n0, n1, n2 = nx[:, 0:1], nx[:, 1:2], nx[:, 2:3]
    q2 = (n0 * n0 + n1 * n1) + n2 * n2
    slot = lax.broadcasted_iota(jnp.int32, (1, NSAMPLE), 1)

    def chunk(k, carry):
        count, acc = carry
        xc = xyz_ref[0, :, pl.ds(k * NC, NC)]
        xc0, xc1, xc2 = xc[0:1, :], xc[1:2, :], xc[2:3, :]
        x2k = (xc0 * xc0 + xc1 * xc1) + xc2 * xc2
        qx = lax.dot_general(nx, xc, (((1,), (0,)), ((), ())),
                             preferred_element_type=jnp.float32)
        d2 = (q2 + x2k) - 2.0 * qx
        mask = (d2 < RADIUS2).astype(jnp.int32)
        cum = mask
        sh = 1
        while sh < NC:
            shifted = jnp.concatenate(
                [jnp.zeros((QT, sh), jnp.int32), cum[:, :-sh]], axis=1)
            cum = cum + shifted
            sh *= 2
        cum = cum + count
        for s in range(NSAMPLE):
            cnt = jnp.sum((cum <= s).astype(jnp.int32), axis=1,
                          keepdims=True)
            acc = acc + cnt * (slot == s).astype(jnp.int32)
        count = cum[:, NC - 1:NC]
        return count, acc

    count0 = jnp.zeros((QT, 1), jnp.int32)
    acc0 = jnp.zeros((QT, NSAMPLE), jnp.int32)
    _, acc = lax.fori_loop(0, NCHUNK, chunk, (count0, acc0))
    first = acc[:, 0:1]
    idx = jnp.where(acc < N, acc, first)
    idx = jnp.where(idx < N, idx, 0)
    idx_ref[0] = idx


def _run_ball(xyz_t, newx, interpret=False):
    return pl.pallas_call(
        _ball_body,
        grid=(B, NPOINT // QT),
        in_specs=[
            pl.BlockSpec((1, 3, N), lambda b, q: (b, 0, 0)),
            pl.BlockSpec((1, QT, 3), lambda b, q: (b, q, 0)),
        ],
        out_specs=pl.BlockSpec((1, QT, NSAMPLE), lambda b, q: (b, q, 0)),
        out_shape=jax.ShapeDtypeStruct((B, NPOINT, NSAMPLE), jnp.int32),
        interpret=interpret,
    )(xyz_t, newx)


NW = 32
QW = NPOINT * B // NW
WPB = NW // B

def _group_body(xyz_hbm, newx_hbm, idx_hbm, out_hbm, xyz_v, nx_v, idx_v, out_v):
    w = lax.axis_index("s") * 2 + lax.axis_index("c")
    b = w // WPB
    q0 = (w % WPB) * QW
    pltpu.sync_copy(xyz_hbm.at[b], xyz_v)
    pltpu.sync_copy(newx_hbm.at[pl.ds(b * NPOINT * 3 + q0 * 3, QW * 3)], nx_v)
    pltpu.sync_copy(
        idx_hbm.at[pl.ds(b * NPOINT * NSAMPLE + q0 * NSAMPLE, QW * NSAMPLE)],
        idx_v)
    lanes = lax.broadcasted_iota(jnp.int32, (16,), 0)

    def body(q, _):
        for c in range(3):
            ctr = plsc.load_gather(nx_v, [jnp.full((16,), q * 3 + c,
                                                   jnp.int32)])
            for h in range(NSAMPLE // 16):
                iv = plsc.load_gather(
                    idx_v, [q * NSAMPLE + h * 16 + lanes])
                g = plsc.load_gather(
                    xyz_v, [jnp.full((16,), c, jnp.int32), iv])
                out_v[pl.ds(c * (QW * NSAMPLE) + q * NSAMPLE + h * 16, 16)] = (
                    g - ctr)
        return 0

    lax.fori_loop(0, QW, body, 0)
    for c in range(3):
        pltpu.sync_copy(
            out_v.at[pl.ds(c * QW * NSAMPLE, QW * NSAMPLE)],
            out_hbm.at[pl.ds((b * 3 + c) * NPOINT * NSAMPLE + q0 * NSAMPLE,
                             QW * NSAMPLE)])


def _run_group(xyz_t, newx_flat, idx_flat, interpret=False):
    mesh = plsc.VectorSubcoreMesh(core_axis_name="c", subcore_axis_name="s")
    f = pl.kernel(
        _group_body,
        out_type=jax.ShapeDtypeStruct((B * 3 * NPOINT * NSAMPLE,), jnp.float32),
        mesh=mesh,
        scratch_types=[
            pltpu.VMEM((3, N), jnp.float32),
            pltpu.VMEM((QW * 3,), jnp.float32),
            pltpu.VMEM((QW * NSAMPLE,), jnp.int32),
            pltpu.VMEM((3 * QW * NSAMPLE,), jnp.float32),
        ],
        compiler_params=pltpu.CompilerParams(needs_layout_passes=False),
        interpret=interpret,
    )
    return f(xyz_t, newx_flat, idx_flat)


def kernel(xyz):
    xyz_t = jnp.transpose(xyz, (0, 2, 1))
    cents = _run_fps(xyz_t)
    newx = jnp.transpose(cents, (1, 0, 2))
    idx = _run_ball(xyz_t, newx)
    newx_flat = newx.reshape(B * NPOINT * 3)
    idx_flat = idx.reshape(B * NPOINT * NSAMPLE)
    grouped = _run_group(xyz_t, newx_flat, idx_flat)
    return grouped.reshape(B, 3, NPOINT, NSAMPLE)


# speedup vs baseline: 9.5545x; 1.0926x over previous
import functools

import jax
import jax.numpy as jnp
from jax import lax
from jax.experimental import pallas as pl
from jax.experimental.pallas import tpu as pltpu
from jax.experimental.pallas import tpu_sc as plsc

B, N, NPOINT, NSAMPLE = 4, 8192, 1024, 32
RADIUS2 = 0.2 * 0.2


def _fps_body(xyz_ref, out_ref):
    x3 = xyz_ref[...]
    lane = lax.broadcasted_iota(jnp.int32, (B, N), 1)

    def body(i, carry):
        dists, far = carry
        onehot = (lane == far)
        zero = jnp.zeros((), jnp.float32)
        c0 = jnp.sum(jnp.where(onehot, x3[:, 0, :], zero), axis=1)
        c1 = jnp.sum(jnp.where(onehot, x3[:, 1, :], zero), axis=1)
        c2 = jnp.sum(jnp.where(onehot, x3[:, 2, :], zero), axis=1)
        cent = jnp.stack([c0, c1, c2], axis=1)
        out_ref[pl.ds(i, 1)] = cent[None]
        diff = x3 - cent[:, :, None]
        d = jnp.sum(diff * diff, axis=1)
        dists = jnp.minimum(dists, d)
        m = jnp.max(dists, axis=1, keepdims=True)
        far = jnp.min(jnp.where(dists == m, lane, N), axis=1,
                      keepdims=True).astype(jnp.int32)
        return dists, far

    dists0 = jnp.full((B, N), 1e10, dtype=jnp.float32)
    far0 = jnp.zeros((B, 1), dtype=jnp.int32)
    lax.fori_loop(0, NPOINT, body, (dists0, far0))


def _run_fps(xyz_t, interpret=False):
    return pl.pallas_call(
        _fps_body,
        out_shape=jax.ShapeDtypeStruct((NPOINT, B, 3), jnp.float32),
        interpret=interpret,
    )(xyz_t)


QT = 256
NC = 512
NCHUNK = N // NC

def _ball_body(xyz_ref, newx_ref, idx_ref):
    nx = newx_ref[0]
    n0, n1, n2 = nx[:, 0:1], nx[:, 1:2], nx[:, 2:3]
    q2 = (n0 * n0 + n1 * n1) + n2 * n2
    slot = lax.broadcasted_iota(jnp.int32, (1, NSAMPLE), 1)

    def chunk(k, carry):
        count, acc = carry

        def active(_):
            xc = xyz_ref[0, :, pl.ds(k * NC, NC)]
            xc0, xc1, xc2 = xc[0:1, :], xc[1:2, :], xc[2:3, :]
            x2k = (xc0 * xc0 + xc1 * xc1) + xc2 * xc2
            qx = lax.dot_general(nx, xc, (((1,), (0,)), ((), ())),
                                 preferred_element_type=jnp.float32)
            d2 = (q2 + x2k) - 2.0 * qx
            mask = (d2 < RADIUS2).astype(jnp.int32)
            cum = mask
            sh = 1
            while sh < NC:
                shifted = jnp.concatenate(
                    [jnp.zeros((QT, sh), jnp.int32), cum[:, :-sh]], axis=1)
                cum = cum + shifted
                sh *= 2
            cum = cum + count
            acc2 = acc
            for s in range(NSAMPLE):
                cnt = jnp.sum((cum <= s).astype(jnp.int32), axis=1,
                              keepdims=True)
                acc2 = acc2 + cnt * (slot == s).astype(jnp.int32)
            return cum[:, NC - 1:NC], acc2

        return lax.cond(jnp.min(count) < NSAMPLE, active,
                        lambda _: (count, acc), 0)

    count0 = jnp.zeros((QT, 1), jnp.int32)
    acc0 = jnp.zeros((QT, NSAMPLE), jnp.int32)
    _, acc = lax.fori_loop(0, NCHUNK, chunk, (count0, acc0))
    first = acc[:, 0:1]
    idx = jnp.where(acc < N, acc, first)
    idx = jnp.where(idx < N, idx, 0)
    idx_ref[0] = idx


def _run_ball(xyz_t, newx, interpret=False):
    return pl.pallas_call(
        _ball_body,
        grid=(B, NPOINT // QT),
        in_specs=[
            pl.BlockSpec((1, 3, N), lambda b, q: (b, 0, 0)),
            pl.BlockSpec((1, QT, 3), lambda b, q: (b, q, 0)),
        ],
        out_specs=pl.BlockSpec((1, QT, NSAMPLE), lambda b, q: (b, q, 0)),
        out_shape=jax.ShapeDtypeStruct((B, NPOINT, NSAMPLE), jnp.int32),
        interpret=interpret,
    )(xyz_t, newx)


NW = 32
QW = NPOINT * B // NW
WPB = NW // B

def _group_body(xyz_hbm, newx_hbm, idx_hbm, out_hbm, xyz_v, nx_v, idx_v, out_v):
    w = lax.axis_index("s") * 2 + lax.axis_index("c")
    b = w // WPB
    q0 = (w % WPB) * QW
    pltpu.sync_copy(xyz_hbm.at[b], xyz_v)
    pltpu.sync_copy(newx_hbm.at[pl.ds(b * NPOINT * 3 + q0 * 3, QW * 3)], nx_v)
    pltpu.sync_copy(
        idx_hbm.at[pl.ds(b * NPOINT * NSAMPLE + q0 * NSAMPLE, QW * NSAMPLE)],
        idx_v)
    lanes = lax.broadcasted_iota(jnp.int32, (16,), 0)

    def body(q, _):
        for c in range(3):
            ctr = plsc.load_gather(nx_v, [jnp.full((16,), q * 3 + c,
                                                   jnp.int32)])
            for h in range(NSAMPLE // 16):
                iv = plsc.load_gather(
                    idx_v, [q * NSAMPLE + h * 16 + lanes])
                g = plsc.load_gather(
                    xyz_v, [jnp.full((16,), c, jnp.int32), iv])
                out_v[pl.ds(c * (QW * NSAMPLE) + q * NSAMPLE + h * 16, 16)] = (
                    g - ctr)
        return 0

    lax.fori_loop(0, QW, body, 0)
    for c in range(3):
        pltpu.sync_copy(
            out_v.at[pl.ds(c * QW * NSAMPLE, QW * NSAMPLE)],
            out_hbm.at[pl.ds((b * 3 + c) * NPOINT * NSAMPLE + q0 * NSAMPLE,
                             QW * NSAMPLE)])


def _run_group(xyz_t, newx_flat, idx_flat, interpret=False):
    mesh = plsc.VectorSubcoreMesh(core_axis_name="c", subcore_axis_name="s")
    f = pl.kernel(
        _group_body,
        out_type=jax.ShapeDtypeStruct((B * 3 * NPOINT * NSAMPLE,), jnp.float32),
        mesh=mesh,
        scratch_types=[
            pltpu.VMEM((3, N), jnp.float32),
            pltpu.VMEM((QW * 3,), jnp.float32),
            pltpu.VMEM((QW * NSAMPLE,), jnp.int32),
            pltpu.VMEM((3 * QW * NSAMPLE,), jnp.float32),
        ],
        compiler_params=pltpu.CompilerParams(needs_layout_passes=False),
        interpret=interpret,
    )
    return f(xyz_t, newx_flat, idx_flat)


def kernel(xyz):
    xyz_t = jnp.transpose(xyz, (0, 2, 1))
    cents = _run_fps(xyz_t)
    newx = jnp.transpose(cents, (1, 0, 2))
    idx = _run_ball(xyz_t, newx)
    newx_flat = newx.reshape(B * NPOINT * 3)
    idx_flat = idx.reshape(B * NPOINT * NSAMPLE)
    grouped = _run_group(xyz_t, newx_flat, idx_flat)
    return grouped.reshape(B, 3, NPOINT, NSAMPLE)


# speedup vs baseline: 10.8335x; 1.1339x over previous
import functools

import jax
import jax.numpy as jnp
from jax import lax
from jax.experimental import pallas as pl
from jax.experimental.pallas import tpu as pltpu
from jax.experimental.pallas import tpu_sc as plsc

B, N, NPOINT, NSAMPLE = 4, 8192, 1024, 32
RADIUS2 = 0.2 * 0.2


def _fps_body(xyz_ref, out_ref):
    x3 = xyz_ref[...]
    x0, x1, x2c = x3[:, 0, :], x3[:, 1, :], x3[:, 2, :]
    lane = lax.broadcasted_iota(jnp.int32, (B, N), 1)

    def body(i, carry):
        dists, far = carry
        onehot = (lane == far)
        zero = jnp.zeros((), jnp.float32)
        c0 = jnp.sum(jnp.where(onehot, x0, zero), axis=1, keepdims=True)
        c1 = jnp.sum(jnp.where(onehot, x1, zero), axis=1, keepdims=True)
        c2 = jnp.sum(jnp.where(onehot, x2c, zero), axis=1, keepdims=True)
        cent = jnp.concatenate([c0, c1, c2], axis=1)
        out_ref[pl.ds(i, 1)] = cent[None]
        d0 = x0 - c0
        d1 = x1 - c1
        d2 = x2c - c2
        d = (d0 * d0 + d2 * d2) + d1 * d1
        dists = jnp.minimum(dists, d)
        m = jnp.max(dists, axis=1, keepdims=True)
        far = jnp.min(jnp.where(dists == m, lane, N), axis=1,
                      keepdims=True).astype(jnp.int32)
        return dists, far

    dists0 = jnp.full((B, N), 1e10, dtype=jnp.float32)
    far0 = jnp.zeros((B, 1), dtype=jnp.int32)
    lax.fori_loop(0, NPOINT, body, (dists0, far0))


def _run_fps(xyz_t, interpret=False):
    return pl.pallas_call(
        _fps_body,
        out_shape=jax.ShapeDtypeStruct((NPOINT, B, 3), jnp.float32),
        interpret=interpret,
    )(xyz_t)


QT = 256
NC = 512
NCHUNK = N // NC

def _ball_body(xyz_ref, newx_ref, idx_ref):
    nx = newx_ref[0]
    n0, n1, n2 = nx[:, 0:1], nx[:, 1:2], nx[:, 2:3]
    q2 = (n0 * n0 + n1 * n1) + n2 * n2
    slot = lax.broadcasted_iota(jnp.int32, (1, NSAMPLE), 1)

    def chunk(k, carry):
        count, acc = carry

        def active(_):
            xc = xyz_ref[0, :, pl.ds(k * NC, NC)]
            xc0, xc1, xc2 = xc[0:1, :], xc[1:2, :], xc[2:3, :]
            x2k = (xc0 * xc0 + xc1 * xc1) + xc2 * xc2
            qx = lax.dot_general(nx, xc, (((1,), (0,)), ((), ())),
                                 preferred_element_type=jnp.float32)
            d2 = (q2 + x2k) - 2.0 * qx
            mask = (d2 < RADIUS2).astype(jnp.int32)
            cum = mask
            sh = 1
            while sh < NC:
                shifted = jnp.concatenate(
                    [jnp.zeros((QT, sh), jnp.int32), cum[:, :-sh]], axis=1)
                cum = cum + shifted
                sh *= 2
            cum = cum + count
            acc2 = acc
            for s in range(NSAMPLE):
                cnt = jnp.sum((cum <= s).astype(jnp.int32), axis=1,
                              keepdims=True)
                acc2 = acc2 + cnt * (slot == s).astype(jnp.int32)
            return cum[:, NC - 1:NC], acc2

        return lax.cond(jnp.min(count) < NSAMPLE, active,
                        lambda _: (count, acc), 0)

    count0 = jnp.zeros((QT, 1), jnp.int32)
    acc0 = jnp.zeros((QT, NSAMPLE), jnp.int32)
    _, acc = lax.fori_loop(0, NCHUNK, chunk, (count0, acc0))
    first = acc[:, 0:1]
    idx = jnp.where(acc < N, acc, first)
    idx = jnp.where(idx < N, idx, 0)
    idx_ref[0] = idx


def _run_ball(xyz_t, newx, interpret=False):
    return pl.pallas_call(
        _ball_body,
        grid=(B, NPOINT // QT),
        in_specs=[
            pl.BlockSpec((1, 3, N), lambda b, q: (b, 0, 0)),
            pl.BlockSpec((1, QT, 3), lambda b, q: (b, q, 0)),
        ],
        out_specs=pl.BlockSpec((1, QT, NSAMPLE), lambda b, q: (b, q, 0)),
        out_shape=jax.ShapeDtypeStruct((B, NPOINT, NSAMPLE), jnp.int32),
        interpret=interpret,
    )(xyz_t, newx)


NW = 32
QW = NPOINT * B // NW
WPB = NW // B

def _group_body(xyz_hbm, newx_hbm, idx_hbm, out_hbm, xyz_v, nx_v, idx_v, out_v):
    w = lax.axis_index("s") * 2 + lax.axis_index("c")
    b = w // WPB
    q0 = (w % WPB) * QW
    pltpu.sync_copy(xyz_hbm.at[b], xyz_v)
    pltpu.sync_copy(newx_hbm.at[pl.ds(b * NPOINT * 3 + q0 * 3, QW * 3)], nx_v)
    pltpu.sync_copy(
        idx_hbm.at[pl.ds(b * NPOINT * NSAMPLE + q0 * NSAMPLE, QW * NSAMPLE)],
        idx_v)
    lanes = lax.broadcasted_iota(jnp.int32, (16,), 0)

    def body(q, _):
        for c in range(3):
            ctr = plsc.load_gather(nx_v, [jnp.full((16,), q * 3 + c,
                                                   jnp.int32)])
            for h in range(NSAMPLE // 16):
                iv = plsc.load_gather(
                    idx_v, [q * NSAMPLE + h * 16 + lanes])
                g = plsc.load_gather(
                    xyz_v, [jnp.full((16,), c, jnp.int32), iv])
                out_v[pl.ds(c * (QW * NSAMPLE) + q * NSAMPLE + h * 16, 16)] = (
                    g - ctr)
        return 0

    lax.fori_loop(0, QW, body, 0)
    for c in range(3):
        pltpu.sync_copy(
            out_v.at[pl.ds(c * QW * NSAMPLE, QW * NSAMPLE)],
            out_hbm.at[pl.ds((b * 3 + c) * NPOINT * NSAMPLE + q0 * NSAMPLE,
                             QW * NSAMPLE)])


def _run_group(xyz_t, newx_flat, idx_flat, interpret=False):
    mesh = plsc.VectorSubcoreMesh(core_axis_name="c", subcore_axis_name="s")
    f = pl.kernel(
        _group_body,
        out_type=jax.ShapeDtypeStruct((B * 3 * NPOINT * NSAMPLE,), jnp.float32),
        mesh=mesh,
        scratch_types=[
            pltpu.VMEM((3, N), jnp.float32),
            pltpu.VMEM((QW * 3,), jnp.float32),
            pltpu.VMEM((QW * NSAMPLE,), jnp.int32),
            pltpu.VMEM((3 * QW * NSAMPLE,), jnp.float32),
        ],
        compiler_params=pltpu.CompilerParams(needs_layout_passes=False),
        interpret=interpret,
    )
    return f(xyz_t, newx_flat, idx_flat)


def kernel(xyz):
    xyz_t = jnp.transpose(xyz, (0, 2, 1))
    cents = _run_fps(xyz_t)
    newx = jnp.transpose(cents, (1, 0, 2))
    idx = _run_ball(xyz_t, newx)
    newx_flat = newx.reshape(B * NPOINT * 3)
    idx_flat = idx.reshape(B * NPOINT * NSAMPLE)
    grouped = _run_group(xyz_t, newx_flat, idx_flat)
    return grouped.reshape(B, 3, NPOINT, NSAMPLE)


# speedup vs baseline: 11.5409x; 1.0653x over previous
import functools

import jax
import jax.numpy as jnp
from jax import lax
from jax.experimental import pallas as pl
from jax.experimental.pallas import tpu as pltpu
from jax.experimental.pallas import tpu_sc as plsc

B, N, NPOINT, NSAMPLE = 4, 8192, 1024, 32
RADIUS2 = 0.2 * 0.2


def _fps_body(xyz_ref, out_ref):
    x3 = xyz_ref[...]
    x0, x1, x2c = x3[:, 0, :], x3[:, 1, :], x3[:, 2, :]
    x12 = jnp.concatenate([x0, x1, x2c], axis=0)
    lane = lax.broadcasted_iota(jnp.int32, (B, N), 1)

    def body(i, carry):
        dists, far = carry
        onehot = (lane == far)
        ohf = onehot.astype(jnp.float32)
        oh12 = jnp.concatenate([ohf, ohf, ohf], axis=0)
        c12 = jnp.sum(oh12 * x12, axis=1, keepdims=True)
        out_ref[pl.ds(i, 1)] = c12.reshape(1, 3 * B)
        c0 = c12[0:B]
        c1 = c12[B:2 * B]
        c2 = c12[2 * B:3 * B]
        d0 = x0 - c0
        d1 = x1 - c1
        d2 = x2c - c2
        d = (d0 * d0 + d2 * d2) + d1 * d1
        dists = jnp.minimum(dists, d)
        m = jnp.max(dists, axis=1, keepdims=True)
        far = jnp.min(jnp.where(dists == m, lane, N), axis=1,
                      keepdims=True).astype(jnp.int32)
        return dists, far

    dists0 = jnp.full((B, N), 1e10, dtype=jnp.float32)
    far0 = jnp.zeros((B, 1), dtype=jnp.int32)
    lax.fori_loop(0, NPOINT, body, (dists0, far0))


def _run_fps(xyz_t, interpret=False):
    c = pl.pallas_call(
        _fps_body,
        out_shape=jax.ShapeDtypeStruct((NPOINT, 3 * B), jnp.float32),
        interpret=interpret,
    )(xyz_t)
    return jnp.transpose(c.reshape(NPOINT, 3, B), (2, 0, 1))


QT = 256
NC = 512
NCHUNK = N // NC

def _ball_body(xyz_ref, newx_ref, idx_ref):
    nx = newx_ref[0]
    n0, n1, n2 = nx[:, 0:1], nx[:, 1:2], nx[:, 2:3]
    q2 = (n0 * n0 + n1 * n1) + n2 * n2
    slot = lax.broadcasted_iota(jnp.int32, (1, NSAMPLE), 1)

    def chunk(k, carry):
        count, acc = carry

        def active(_):
            xc = xyz_ref[0, :, pl.ds(k * NC, NC)]
            xc0, xc1, xc2 = xc[0:1, :], xc[1:2, :], xc[2:3, :]
            x2k = (xc0 * xc0 + xc1 * xc1) + xc2 * xc2
            qx = lax.dot_general(nx, xc, (((1,), (0,)), ((), ())),
                                 preferred_element_type=jnp.float32)
            d2 = (q2 + x2k) - 2.0 * qx
            mask = (d2 < RADIUS2).astype(jnp.int32)
            cum = mask
            sh = 1
            while sh < NC:
                shifted = jnp.concatenate(
                    [jnp.zeros((QT, sh), jnp.int32), cum[:, :-sh]], axis=1)
                cum = cum + shifted
                sh *= 2
            cum = cum + count
            acc2 = acc
            for s in range(NSAMPLE):
                cnt = jnp.sum((cum <= s).astype(jnp.int32), axis=1,
                              keepdims=True)
                acc2 = acc2 + cnt * (slot == s).astype(jnp.int32)
            return cum[:, NC - 1:NC], acc2

        return lax.cond(jnp.min(count) < NSAMPLE, active,
                        lambda _: (count, acc), 0)

    count0 = jnp.zeros((QT, 1), jnp.int32)
    acc0 = jnp.zeros((QT, NSAMPLE), jnp.int32)
    _, acc = lax.fori_loop(0, NCHUNK, chunk, (count0, acc0))
    first = acc[:, 0:1]
    idx = jnp.where(acc < N, acc, first)
    idx = jnp.where(idx < N, idx, 0)
    idx_ref[0] = idx


def _run_ball(xyz_t, newx, interpret=False):
    return pl.pallas_call(
        _ball_body,
        grid=(B, NPOINT // QT),
        in_specs=[
            pl.BlockSpec((1, 3, N), lambda b, q: (b, 0, 0)),
            pl.BlockSpec((1, QT, 3), lambda b, q: (b, q, 0)),
        ],
        out_specs=pl.BlockSpec((1, QT, NSAMPLE), lambda b, q: (b, q, 0)),
        out_shape=jax.ShapeDtypeStruct((B, NPOINT, NSAMPLE), jnp.int32),
        interpret=interpret,
    )(xyz_t, newx)


NW = 32
QW = NPOINT * B // NW
WPB = NW // B

def _group_body(xyz_hbm, newx_hbm, idx_hbm, out_hbm, xyz_v, nx_v, idx_v, out_v):
    w = lax.axis_index("s") * 2 + lax.axis_index("c")
    b = w // WPB
    q0 = (w % WPB) * QW
    pltpu.sync_copy(xyz_hbm.at[b], xyz_v)
    pltpu.sync_copy(newx_hbm.at[pl.ds(b * NPOINT * 3 + q0 * 3, QW * 3)], nx_v)
    pltpu.sync_copy(
        idx_hbm.at[pl.ds(b * NPOINT * NSAMPLE + q0 * NSAMPLE, QW * NSAMPLE)],
        idx_v)
    lanes = lax.broadcasted_iota(jnp.int32, (16,), 0)

    def body(q, _):
        for c in range(3):
            ctr = plsc.load_gather(nx_v, [jnp.full((16,), q * 3 + c,
                                                   jnp.int32)])
            for h in range(NSAMPLE // 16):
                iv = plsc.load_gather(
                    idx_v, [q * NSAMPLE + h * 16 + lanes])
                g = plsc.load_gather(
                    xyz_v, [jnp.full((16,), c, jnp.int32), iv])
                out_v[pl.ds(c * (QW * NSAMPLE) + q * NSAMPLE + h * 16, 16)] = (
                    g - ctr)
        return 0

    lax.fori_loop(0, QW, body, 0)
    for c in range(3):
        pltpu.sync_copy(
            out_v.at[pl.ds(c * QW * NSAMPLE, QW * NSAMPLE)],
            out_hbm.at[pl.ds((b * 3 + c) * NPOINT * NSAMPLE + q0 * NSAMPLE,
                             QW * NSAMPLE)])


def _run_group(xyz_t, newx_flat, idx_flat, interpret=False):
    mesh = plsc.VectorSubcoreMesh(core_axis_name="c", subcore_axis_name="s")
    f = pl.kernel(
        _group_body,
        out_type=jax.ShapeDtypeStruct((B * 3 * NPOINT * NSAMPLE,), jnp.float32),
        mesh=mesh,
        scratch_types=[
            pltpu.VMEM((3, N), jnp.float32),
            pltpu.VMEM((QW * 3,), jnp.float32),
            pltpu.VMEM((QW * NSAMPLE,), jnp.int32),
            pltpu.VMEM((3 * QW * NSAMPLE,), jnp.float32),
        ],
        compiler_params=pltpu.CompilerParams(needs_layout_passes=False),
        interpret=interpret,
    )
    return f(xyz_t, newx_flat, idx_flat)


def kernel(xyz):
    xyz_t = jnp.transpose(xyz, (0, 2, 1))
    newx = _run_fps(xyz_t)
    idx = _run_ball(xyz_t, newx)
    newx_flat = newx.reshape(B * NPOINT * 3)
    idx_flat = idx.reshape(B * NPOINT * NSAMPLE)
    grouped = _run_group(xyz_t, newx_flat, idx_flat)
    return grouped.reshape(B, 3, NPOINT, NSAMPLE)
